# baseline (speedup 1.0000x reference)
import functools

import jax
import jax.numpy as jnp
from jax import lax
from jax.experimental import pallas as pl
from jax.experimental.pallas import tpu as pltpu
from jax.experimental.pallas import tpu_sc as plsc


def kernel(x, pe):
    T = x.shape[1]
    D = pe.shape[2]
    n = T * D

    info = plsc.get_sparse_core_info()
    nw = info.num_cores * info.num_subcores
    assert n % nw == 0 and (n // nw) % 8 == 0
    chunk = n // nw

    mesh = plsc.VectorSubcoreMesh(core_axis_name="c", subcore_axis_name="s")

    @functools.partial(
        pl.kernel,
        mesh=mesh,
        out_type=jax.ShapeDtypeStruct((n,), pe.dtype),
    )
    def sc_copy(pe_hbm, out_hbm):
        wid = lax.axis_index("s") * info.num_cores + lax.axis_index("c")

        @pl.when(wid == 0)
        def _():
            pltpu.sync_copy(pe_hbm.at[pl.ds(0, n)], out_hbm)

    out = sc_copy(pe.reshape(-1))
    return out.reshape(1, T, D)


# speedup vs baseline: 5.4681x; 5.4681x over previous
import jax
import jax.numpy as jnp
from jax.experimental import pallas as pl
from jax.experimental.pallas import tpu as pltpu


def _slice_copy(pe_hbm, o_hbm, sem):
    T = o_hbm.shape[1]
    pltpu.make_async_copy(pe_hbm.at[:, pl.ds(0, T), :], o_hbm, sem).start()
    pltpu.make_async_copy(pe_hbm.at[:, pl.ds(0, T), :], o_hbm, sem).wait()


def kernel(x, pe):
    T = x.shape[1]
    D = pe.shape[2]
    return pl.pallas_call(
        _slice_copy,
        out_shape=jax.ShapeDtypeStruct((1, T, D), pe.dtype),
        in_specs=[pl.BlockSpec(memory_space=pl.ANY)],
        out_specs=pl.BlockSpec(memory_space=pl.ANY),
        scratch_shapes=[pltpu.SemaphoreType.DMA],
    )(pe)


# speedup vs baseline: 14.7272x; 2.6933x over previous
import jax
import jax.numpy as jnp
from jax.experimental import pallas as pl


def _slice_copy(pe_ref, o_ref):
    o_ref[...] = pe_ref[...]


def kernel(x, pe):
    T = x.shape[1]
    D = pe.shape[2]
    return pl.pallas_call(
        _slice_copy,
        grid=(1,),
        out_shape=jax.ShapeDtypeStruct((1, T, D), pe.dtype),
        in_specs=[pl.BlockSpec((1, T, D), lambda i: (0, 0, 0))],
        out_specs=pl.BlockSpec((1, T, D), lambda i: (0, 0, 0)),
    )(pe)


# speedup vs baseline: 14.8028x; 1.0051x over previous
import jax
import jax.numpy as jnp
from jax.experimental import pallas as pl
from jax.experimental.pallas import tpu as pltpu


def _slice_copy(pe_hbm, o_ref, sem):
    T = o_ref.shape[1]
    cp = pltpu.make_async_copy(pe_hbm.at[:, pl.ds(0, T), :], o_ref, sem)
    cp.start()
    cp.wait()


def kernel(x, pe):
    T = x.shape[1]
    D = pe.shape[2]
    return pl.pallas_call(
        _slice_copy,
        grid=(1,),
        out_shape=jax.ShapeDtypeStruct((1, T, D), pe.dtype),
        in_specs=[pl.BlockSpec(memory_space=pl.ANY)],
        out_specs=pl.BlockSpec((1, T, D), lambda i: (0, 0, 0)),
        scratch_shapes=[pltpu.SemaphoreType.DMA],
    )(pe)
